# baseline (speedup 1.0000x reference)
import functools

import jax
import jax.numpy as jnp
from jax import lax
from jax.experimental import pallas as pl
from jax.experimental.pallas import tpu as pltpu
from jax.experimental.pallas import tpu_sc as plsc

N = 10000
D = 128
H = 16
E = 320000
NP = 10240
NC = 2
NS = 16
NW = NC * NS
B = 128
K = 8
SUP = 10
CH = K * SUP
EP = NW * CH * B
ER = EP // B
RPT = NP // NS


def _prep_body(w_ref, x_ref, wl_ref, wr_ref, a0_ref, a1_ref, dst_ref,
               y1_ref, xr_ref, dstm_ref):
    x = x_ref[...]
    y1_ref[...] = jnp.dot(x, wl_ref[...], preferred_element_type=jnp.float32)
    xr_ref[...] = jnp.dot(x, wr_ref[...], preferred_element_type=jnp.float32)
    me = w_ref[0] * a0_ref[...] + w_ref[1] * a1_ref[...]
    dstm_ref[...] = jnp.where(me != 0.0, dst_ref[...], jnp.int32(N))


_prep = pl.pallas_call(
    _prep_body,
    grid=(10,),
    in_specs=[
        pl.BlockSpec(memory_space=pltpu.SMEM),
        pl.BlockSpec((NP // 10, D), lambda i: (i, 0)),
        pl.BlockSpec((D, D), lambda i: (0, 0)),
        pl.BlockSpec((D, D), lambda i: (0, 0)),
        pl.BlockSpec((ER // 10, B), lambda i: (i, 0)),
        pl.BlockSpec((ER // 10, B), lambda i: (i, 0)),
        pl.BlockSpec((ER // 10, B), lambda i: (i, 0)),
    ],
    out_specs=[
        pl.BlockSpec((NP // 10, D), lambda i: (i, 0)),
        pl.BlockSpec((NP // 10, D), lambda i: (i, 0)),
        pl.BlockSpec((ER // 10, B), lambda i: (i, 0)),
    ],
    out_shape=[
        jax.ShapeDtypeStruct((NP, D), jnp.float32),
        jax.ShapeDtypeStruct((NP, D), jnp.float32),
        jax.ShapeDtypeStruct((ER, B), jnp.int32),
    ],
)


def _pass1_body(y1_hbm, xr_hbm, src_hbm, dst_hbm, out_hbm,
                src_v, dst_v, rows_v, zb_v,
                acc_sh, tab_sh, sem_a, sem_b):
    c = lax.axis_index("c")
    s = lax.axis_index("s")
    wid = s * NC + c

    pltpu.sync_copy(y1_hbm.at[pl.ds(s * RPT, RPT), pl.ds(0, H)],
                    tab_sh.at[pl.ds(s * RPT, RPT)])

    @pl.when(c == 0)
    def _():
        pltpu.sync_copy(xr_hbm.at[pl.ds(s * RPT, RPT), pl.ds(0, H)],
                        acc_sh.at[pl.ds(s * RPT, RPT)])

    @pl.when(c != 0)
    def _():
        def _zero_row(i, carry):
            zb_v[i, :] = jnp.zeros((H,), jnp.float32)
            return carry
        lax.fori_loop(0, B, _zero_row, 0)
        for k in range(RPT // B):
            pltpu.sync_copy(zb_v, acc_sh.at[pl.ds(s * RPT + k * B, B)])

    pltpu.sync_copy(src_hbm.at[pl.ds(wid * CH, CH)], src_v)
    pltpu.sync_copy(dst_hbm.at[pl.ds(wid * CH, CH)], dst_v)
    plsc.subcore_barrier()

    sems = (sem_a, sem_b)

    def _fire(sup, buf):
        handles = []
        for b in range(K):
            handles.append(pltpu.async_copy(
                tab_sh.at[src_v.at[sup * K + b]],
                rows_v.at[pl.ds((buf * K + b) * B, B)],
                sems[buf]))
        return handles

    handles = _fire(0, 0)
    for sup in range(SUP):
        nxt = _fire(sup + 1, (sup + 1) % 2) if sup + 1 < SUP else None
        for b in range(K):
            handles[b].wait()
            pltpu.sync_copy(
                rows_v.at[pl.ds(((sup % 2) * K + b) * B, B)],
                acc_sh.at[dst_v.at[sup * K + b]], add=True)
        handles = nxt

    plsc.subcore_barrier()
    pltpu.sync_copy(acc_sh.at[pl.ds(s * RPT, RPT)],
                    out_hbm.at[c, pl.ds(s * RPT, RPT)])


_pass1 = pl.kernel(
    _pass1_body,
    out_type=jax.ShapeDtypeStruct((NC, NP, H), jnp.float32),
    mesh=plsc.VectorSubcoreMesh(core_axis_name="c", subcore_axis_name="s"),
    scratch_types=[
        pltpu.VMEM((CH, B), jnp.int32),
        pltpu.VMEM((CH, B), jnp.int32),
        pltpu.VMEM((2 * K * B, H), jnp.float32),
        pltpu.VMEM((B, H), jnp.float32),
        pltpu.VMEM_SHARED((NP, H), jnp.float32),
        pltpu.VMEM_SHARED((NP, H), jnp.float32),
        pltpu.SemaphoreType.DMA,
        pltpu.SemaphoreType.DMA,
    ],
    compiler_params=pltpu.CompilerParams(use_tc_tiling_on_sc=False,
                                         needs_layout_passes=False),
)


NG = RPT // H


def _pass2_body(p_hbm, src_hbm, dst_hbm, wb_hbm, out_hbm,
                src_v, dst_v, rows_v, p0_v, p1_v, y2_v, hr_v, q_v, wb_v,
                acc_sh, tab_sh, sem_a, sem_b):
    c = lax.axis_index("c")
    s = lax.axis_index("s")
    wid = s * NC + c

    pltpu.sync_copy(p_hbm.at[0, pl.ds(s * RPT, RPT)], p0_v)
    pltpu.sync_copy(p_hbm.at[1, pl.ds(s * RPT, RPT)], p1_v)
    pltpu.sync_copy(wb_hbm, wb_v)
    b1v = wb_v[0, :]
    w2l = wb_v[1, :]
    w2r = wb_v[2, :]
    b2v = wb_v[3, :]

    def _group(g, carry):
        row_idx = g * H + lax.iota(jnp.int32, H)
        y2acc = jnp.zeros((H,), jnp.float32)
        hracc = jnp.zeros((H,), jnp.float32)
        for f in range(H):
            col_idx = jnp.full((H,), f, jnp.int32)
            c0 = plsc.load_gather(p0_v, [row_idx, col_idx])
            c1 = plsc.load_gather(p1_v, [row_idx, col_idx])
            hcol = jnp.maximum(c0 + c1 + b1v[f], 0.0)
            y2acc = y2acc + hcol * w2l[f]
            hracc = hracc + hcol * w2r[f]
        y2_v[pl.ds(g * H, H)] = y2acc
        hr_v[pl.ds(g * H, H)] = hracc + b2v
        return carry
    lax.fori_loop(0, NG, _group, 0)
    pltpu.sync_copy(y2_v, tab_sh.at[pl.ds(s * RPT, RPT)])

    def _zero_g(g, carry):
        y2_v[pl.ds(g * H, H)] = jnp.zeros((H,), jnp.float32)
        return carry
    lax.fori_loop(0, NG, _zero_g, 0)
    pltpu.sync_copy(y2_v, acc_sh.at[pl.ds(s * RPT, RPT)])
    plsc.subcore_barrier()

    pltpu.sync_copy(src_hbm.at[pl.ds(wid * CH, CH)], src_v)
    pltpu.sync_copy(dst_hbm.at[pl.ds(wid * CH, CH)], dst_v)

    sems = (sem_a, sem_b)

    def _fire(sup, buf):
        handles = []
        for b in range(K):
            handles.append(pltpu.async_copy(
                tab_sh.at[src_v.at[sup * K + b]],
                rows_v.at[pl.ds((buf * K + b) * B, B)],
                sems[buf]))
        return handles

    handles = _fire(0, 0)
    for sup in range(SUP):
        nxt = _fire(sup + 1, (sup + 1) % 2) if sup + 1 < SUP else None
        for b in range(K):
            handles[b].wait()
            pltpu.sync_copy(
                rows_v.at[pl.ds(((sup % 2) * K + b) * B, B)],
                acc_sh.at[dst_v.at[sup * K + b]], add=True)
        handles = nxt

    plsc.subcore_barrier()

    pltpu.sync_copy(acc_sh.at[pl.ds(s * RPT, RPT)], q_v)

    def _combine(g, carry):
        base = pl.ds(g * H, H)
        contrib = jnp.where(c == 0, hr_v[base], jnp.zeros((H,), jnp.float32))
        q_v[base] = q_v[base] + contrib
        return carry
    lax.fori_loop(0, NG, _combine, 0)
    pltpu.sync_copy(q_v, out_hbm.at[c, pl.ds(s * RPT, RPT)])


_pass2 = pl.kernel(
    _pass2_body,
    out_type=jax.ShapeDtypeStruct((NC, NP), jnp.float32),
    mesh=plsc.VectorSubcoreMesh(core_axis_name="c", subcore_axis_name="s"),
    scratch_types=[
        pltpu.VMEM((CH, B), jnp.int32),
        pltpu.VMEM((CH, B), jnp.int32),
        pltpu.VMEM((2 * K * B,), jnp.float32),
        pltpu.VMEM((RPT, H), jnp.float32),
        pltpu.VMEM((RPT, H), jnp.float32),
        pltpu.VMEM((RPT,), jnp.float32),
        pltpu.VMEM((RPT,), jnp.float32),
        pltpu.VMEM((RPT,), jnp.float32),
        pltpu.VMEM((8, H), jnp.float32),
        pltpu.VMEM_SHARED((NP,), jnp.float32),
        pltpu.VMEM_SHARED((NP,), jnp.float32),
        pltpu.SemaphoreType.DMA,
        pltpu.SemaphoreType.DMA,
    ],
    compiler_params=pltpu.CompilerParams(use_tc_tiling_on_sc=False,
                                         needs_layout_passes=False),
)


def _final_body(o_ref, out_ref):
    out_ref[...] = o_ref[0] + o_ref[1]


_final = pl.pallas_call(
    _final_body,
    grid=(1,),
    in_specs=[pl.BlockSpec((NC, NP // 128, 128), lambda i: (0, 0, 0))],
    out_specs=pl.BlockSpec((NP // 128, 128), lambda i: (0, 0)),
    out_shape=jax.ShapeDtypeStruct((NP // 128, 128), jnp.float32),
)


def kernel(x, edge_index, A0, A1, w, W1l, W1r, b1, W2l, W2r, b2):
    erv = E // B
    ei = edge_index.reshape(2 * E)

    def _rows(v, fill):
        return jnp.pad(v.reshape(erv, B), ((0, ER - erv), (0, 0)),
                       constant_values=fill)

    src2 = _rows(ei[:E], 0)
    dst2 = _rows(ei[E:], N)
    a02 = _rows(A0, 0.0)
    a12 = _rows(A1, 0.0)
    xp = jnp.pad(x, ((0, NP - N), (0, 0)))
    w1lp = jnp.pad(W1l, ((0, 0), (0, D - H)))
    w1rp = jnp.pad(W1r, ((0, 0), (0, D - H)))
    wb = jnp.stack([b1, W2l.reshape(H), W2r.reshape(H),
                    jnp.broadcast_to(b2, (H,)), jnp.pad(w, (0, H - 2)),
                    jnp.zeros((H,), jnp.float32), jnp.zeros((H,), jnp.float32),
                    jnp.zeros((H,), jnp.float32)])

    y1, xr, dstm = _prep(w, xp, w1lp, w1rp, a02, a12, dst2)
    p = _pass1(y1, xr, src2, dstm)
    o = _pass2(p, src2, dstm, wb)
    out = _final(o.reshape(NC, NP // 128, 128))
    return out.reshape(NP, 1)[:N]


# speedup vs baseline: 1.2129x; 1.2129x over previous
import jax
import jax.numpy as jnp
from jax import lax
from jax.experimental import pallas as pl
from jax.experimental.pallas import tpu as pltpu
from jax.experimental.pallas import tpu_sc as plsc

N = 10000
D = 128
H = 16
E = 320000
NP = 10240
NC = 2
NS = 16
NW = NC * NS
B = 128
NR = E // B
K = 6
SUP = 13
CW = K * SUP
XW = NR - NW * CW
CH = CW + 1
RPT = NP // NS
NG = RPT // H


def _prep_body(x_ref, wl_ref, wr_ref, y1_ref, xr_ref):
    x = x_ref[...]
    y1_ref[...] = jnp.dot(x, wl_ref[...], preferred_element_type=jnp.float32)
    xr_ref[...] = jnp.dot(x, wr_ref[...], preferred_element_type=jnp.float32)


_prep = pl.pallas_call(
    _prep_body,
    grid=(10,),
    in_specs=[
        pl.BlockSpec((NP // 10, D), lambda i: (i, 0)),
        pl.BlockSpec((D, D), lambda i: (0, 0)),
        pl.BlockSpec((D, D), lambda i: (0, 0)),
    ],
    out_specs=[
        pl.BlockSpec((NP // 10, D), lambda i: (i, 0)),
        pl.BlockSpec((NP // 10, D), lambda i: (i, 0)),
    ],
    out_shape=[
        jax.ShapeDtypeStruct((NP, D), jnp.float32),
        jax.ShapeDtypeStruct((NP, D), jnp.float32),
    ],
)


def _worker_rows(wid):
    base = wid * CW + jnp.minimum(wid, XW)
    return base, wid < XW


def _stage_edges(ei_hbm, half, base, extra, v):
    pltpu.sync_copy(ei_hbm.at[pl.ds(base, CW), half], v.at[pl.ds(0, CW)])

    @pl.when(extra)
    def _():
        pltpu.sync_copy(ei_hbm.at[pl.ds(base + CW, 1), half],
                        v.at[pl.ds(CW, 1)])


def _stage_rows(src_hbm, base, extra, v):
    pltpu.sync_copy(src_hbm.at[pl.ds(base, CW)], v.at[pl.ds(0, CW)])

    @pl.when(extra)
    def _():
        pltpu.sync_copy(src_hbm.at[pl.ds(base + CW, 1)], v.at[pl.ds(CW, 1)])


def _edge_loop(tab_sh, acc_sh, src_v, dst_v, rows_v, sems, extra):
    def _sl(pos):
        return pl.ds(pos * B, B)

    def _fire(sup, buf):
        handles = []
        for r in range(K):
            handles.append(pltpu.async_copy(
                tab_sh.at[src_v.at[sup * K + r]],
                rows_v.at[_sl(buf * K + r)], sems[buf]))
        return handles

    handles = _fire(0, 0)
    for sup in range(SUP):
        nxt = _fire(sup + 1, (sup + 1) % 2) if sup + 1 < SUP else None
        for r in range(K):
            handles[r].wait()
            pltpu.sync_copy(rows_v.at[_sl((sup % 2) * K + r)],
                            acc_sh.at[dst_v.at[sup * K + r]], add=True)
        handles = nxt

    @pl.when(extra)
    def _():
        pltpu.async_copy(tab_sh.at[src_v.at[CW]], rows_v.at[_sl(0)],
                         sems[0]).wait()
        pltpu.sync_copy(rows_v.at[_sl(0)], acc_sh.at[dst_v.at[CW]], add=True)


def _pass1_body(y1_hbm, xr_hbm, ei_hbm, a0_hbm, a1_hbm, wb_hbm,
                out_hbm, dstm_hbm,
                src_v, dst_v, a0_v, a1_v, wb_v, rows_v, zb_v,
                acc_sh, tab_sh, sem_a, sem_b):
    c = lax.axis_index("c")
    s = lax.axis_index("s")
    wid = s * NC + c
    base, extra = _worker_rows(wid)

    pltpu.sync_copy(y1_hbm.at[pl.ds(s * RPT, RPT), pl.ds(0, H)],
                    tab_sh.at[pl.ds(s * RPT, RPT)])

    @pl.when(c == 0)
    def _():
        pltpu.sync_copy(xr_hbm.at[pl.ds(s * RPT, RPT), pl.ds(0, H)],
                        acc_sh.at[pl.ds(s * RPT, RPT)])

    @pl.when(c != 0)
    def _():
        def _zero_row(i, carry):
            zb_v[i, :] = jnp.zeros((H,), jnp.float32)
            return carry
        lax.fori_loop(0, B, _zero_row, 0)
        for k in range(RPT // B):
            pltpu.sync_copy(zb_v, acc_sh.at[pl.ds(s * RPT + k * B, B)])

    _stage_edges(ei_hbm, 0, base, extra, src_v)
    _stage_edges(ei_hbm, 1, base, extra, dst_v)
    _stage_rows(a0_hbm, base, extra, a0_v)
    _stage_rows(a1_hbm, base, extra, a1_v)
    pltpu.sync_copy(wb_hbm, wb_v)
    wv = wb_v[4, :]
    w0 = wv[0]
    w1 = wv[1]

    def _mask_row(j, carry):
        for k in range(B // H):
            sl = pl.ds(k * H, H)
            me = w0 * a0_v[j, sl] + w1 * a1_v[j, sl]
            dst_v[j, sl] = jnp.where(me != 0.0, dst_v[j, sl], jnp.int32(N))
        return carry
    lax.fori_loop(0, CH, _mask_row, 0)

    pltpu.sync_copy(dst_v.at[pl.ds(0, CW)], dstm_hbm.at[pl.ds(base, CW)])

    @pl.when(extra)
    def _():
        pltpu.sync_copy(dst_v.at[pl.ds(CW, 1)],
                        dstm_hbm.at[pl.ds(base + CW, 1)])

    plsc.subcore_barrier()
    _edge_loop(tab_sh, acc_sh, src_v, dst_v, rows_v, (sem_a, sem_b), extra)
    plsc.subcore_barrier()
    pltpu.sync_copy(acc_sh.at[pl.ds(s * RPT, RPT)],
                    out_hbm.at[c, pl.ds(s * RPT, RPT)])


_pass1 = pl.kernel(
    _pass1_body,
    out_type=[jax.ShapeDtypeStruct((NC, NP, H), jnp.float32),
              jax.ShapeDtypeStruct((NR, B), jnp.int32)],
    mesh=plsc.VectorSubcoreMesh(core_axis_name="c", subcore_axis_name="s"),
    scratch_types=[
        pltpu.VMEM((CH, B), jnp.int32),
        pltpu.VMEM((CH, B), jnp.int32),
        pltpu.VMEM((CH, B), jnp.float32),
        pltpu.VMEM((CH, B), jnp.float32),
        pltpu.VMEM((8, H), jnp.float32),
        pltpu.VMEM((2 * K * B, H), jnp.float32),
        pltpu.VMEM((B, H), jnp.float32),
        pltpu.VMEM_SHARED((NP, H), jnp.float32),
        pltpu.VMEM_SHARED((NP, H), jnp.float32),
        pltpu.SemaphoreType.DMA,
        pltpu.SemaphoreType.DMA,
    ],
    compiler_params=pltpu.CompilerParams(use_tc_tiling_on_sc=False,
                                         needs_layout_passes=False),
)


def _pass2_body(p_hbm, ei_hbm, dstm_hbm, wb_hbm, out_hbm,
                src_v, dst_v, rows_v, p0_v, p1_v, y2_v, hr_v, q_v, wb_v,
                acc_sh, tab_sh, sem_a, sem_b):
    c = lax.axis_index("c")
    s = lax.axis_index("s")
    wid = s * NC + c
    base, extra = _worker_rows(wid)

    pltpu.sync_copy(p_hbm.at[0, pl.ds(s * RPT, RPT)], p0_v)
    pltpu.sync_copy(p_hbm.at[1, pl.ds(s * RPT, RPT)], p1_v)
    pltpu.sync_copy(wb_hbm, wb_v)
    b1v = wb_v[0, :]
    w2l = wb_v[1, :]
    w2r = wb_v[2, :]
    b2v = wb_v[3, :]

    def _group(g, carry):
        row_idx = g * H + lax.iota(jnp.int32, H)
        y2acc = jnp.zeros((H,), jnp.float32)
        hracc = jnp.zeros((H,), jnp.float32)
        for f in range(H):
            col_idx = jnp.full((H,), f, jnp.int32)
            c0 = plsc.load_gather(p0_v, [row_idx, col_idx])
            c1 = plsc.load_gather(p1_v, [row_idx, col_idx])
            hcol = jnp.maximum(c0 + c1 + b1v[f], 0.0)
            y2acc = y2acc + hcol * w2l[f]
            hracc = hracc + hcol * w2r[f]
        y2_v[pl.ds(g * H, H)] = y2acc
        hr_v[pl.ds(g * H, H)] = hracc + b2v
        return carry
    lax.fori_loop(0, NG, _group, 0)
    pltpu.sync_copy(y2_v, tab_sh.at[pl.ds(s * RPT, RPT)])

    def _zero_g(g, carry):
        y2_v[pl.ds(g * H, H)] = jnp.zeros((H,), jnp.float32)
        return carry
    lax.fori_loop(0, NG, _zero_g, 0)
    pltpu.sync_copy(y2_v, acc_sh.at[pl.ds(s * RPT, RPT)])

    _stage_edges(ei_hbm, 0, base, extra, src_v)
    _stage_rows(dstm_hbm, base, extra, dst_v)
    plsc.subcore_barrier()

    _edge_loop(tab_sh, acc_sh, src_v, dst_v, rows_v, (sem_a, sem_b), extra)
    plsc.subcore_barrier()

    pltpu.sync_copy(acc_sh.at[pl.ds(s * RPT, RPT)], q_v)

    def _combine(g, carry):
        sl = pl.ds(g * H, H)
        contrib = jnp.where(c == 0, hr_v[sl], jnp.zeros((H,), jnp.float32))
        q_v[sl] = q_v[sl] + contrib
        return carry
    lax.fori_loop(0, NG, _combine, 0)
    pltpu.sync_copy(q_v, out_hbm.at[c, pl.ds(s * RPT, RPT)])


_pass2 = pl.kernel(
    _pass2_body,
    out_type=jax.ShapeDtypeStruct((NC, NP), jnp.float32),
    mesh=plsc.VectorSubcoreMesh(core_axis_name="c", subcore_axis_name="s"),
    scratch_types=[
        pltpu.VMEM((CH, B), jnp.int32),
        pltpu.VMEM((CH, B), jnp.int32),
        pltpu.VMEM((2 * K * B,), jnp.float32),
        pltpu.VMEM((RPT, H), jnp.float32),
        pltpu.VMEM((RPT, H), jnp.float32),
        pltpu.VMEM((RPT,), jnp.float32),
        pltpu.VMEM((RPT,), jnp.float32),
        pltpu.VMEM((RPT,), jnp.float32),
        pltpu.VMEM((8, H), jnp.float32),
        pltpu.VMEM_SHARED((NP,), jnp.float32),
        pltpu.VMEM_SHARED((NP,), jnp.float32),
        pltpu.SemaphoreType.DMA,
        pltpu.SemaphoreType.DMA,
    ],
    compiler_params=pltpu.CompilerParams(use_tc_tiling_on_sc=False,
                                         needs_layout_passes=False),
)


def _final_body(o_ref, out_ref):
    out_ref[...] = o_ref[0] + o_ref[1]


_final = pl.pallas_call(
    _final_body,
    grid=(1,),
    in_specs=[pl.BlockSpec((NC, NP // 128, 128), lambda i: (0, 0, 0))],
    out_specs=pl.BlockSpec((NP // 128, 128), lambda i: (0, 0)),
    out_shape=jax.ShapeDtypeStruct((NP // 128, 128), jnp.float32),
)


def kernel(x, edge_index, A0, A1, w, W1l, W1r, b1, W2l, W2r, b2):
    ei3 = jnp.transpose(edge_index.reshape(2, NR, B), (1, 0, 2))
    a02 = A0.reshape(NR, B)
    a12 = A1.reshape(NR, B)
    xp = jnp.pad(x, ((0, NP - N), (0, 0)))
    w1lp = jnp.pad(W1l, ((0, 0), (0, D - H)))
    w1rp = jnp.pad(W1r, ((0, 0), (0, D - H)))
    wb = jnp.stack([b1, W2l.reshape(H), W2r.reshape(H),
                    jnp.broadcast_to(b2, (H,)), jnp.pad(w, (0, H - 2)),
                    jnp.zeros((H,), jnp.float32), jnp.zeros((H,), jnp.float32),
                    jnp.zeros((H,), jnp.float32)])

    y1, xr = _prep(xp, w1lp, w1rp)
    p, dstm = _pass1(y1, xr, ei3, a02, a12, wb)
    o = _pass2(p, ei3, dstm, wb)
    out = _final(o.reshape(NC, NP // 128, 128))
    return out.reshape(NP, 1)[:N]


# speedup vs baseline: 1.2271x; 1.0117x over previous
import jax
import jax.numpy as jnp
from jax import lax
from jax.experimental import pallas as pl
from jax.experimental.pallas import tpu as pltpu
from jax.experimental.pallas import tpu_sc as plsc

N = 10000
D = 128
H = 16
E = 320000
NP = 10240
NC = 2
NS = 16
NW = NC * NS
B = 128
NR = E // B
K = 6
SUP = 13
CW = K * SUP
XW = NR - NW * CW
CH = CW + 1
RPT = NP // NS
NG = RPT // H


def _prep_body(x_ref, w_ref, yx_ref):
    x = x_ref[...]
    yx_ref[...] = jnp.dot(x, w_ref[...], preferred_element_type=jnp.float32)


_prep = pl.pallas_call(
    _prep_body,
    grid=(10,),
    in_specs=[
        pl.BlockSpec((NP // 10, D), lambda i: (i, 0)),
        pl.BlockSpec((D, D), lambda i: (0, 0)),
    ],
    out_specs=pl.BlockSpec((NP // 10, D), lambda i: (i, 0)),
    out_shape=jax.ShapeDtypeStruct((NP, D), jnp.float32),
)


def _worker_rows(wid):
    base = wid * CW + jnp.minimum(wid, XW)
    return base, wid < XW


def _stage_edges(ei_hbm, half, base, extra, v):
    pltpu.sync_copy(ei_hbm.at[pl.ds(base, CW), half], v.at[pl.ds(0, CW)])

    @pl.when(extra)
    def _():
        pltpu.sync_copy(ei_hbm.at[pl.ds(base + CW, 1), half],
                        v.at[pl.ds(CW, 1)])


def _stage_rows(src_hbm, base, extra, v):
    pltpu.sync_copy(src_hbm.at[pl.ds(base, CW)], v.at[pl.ds(0, CW)])

    @pl.when(extra)
    def _():
        pltpu.sync_copy(src_hbm.at[pl.ds(base + CW, 1)], v.at[pl.ds(CW, 1)])


def _edge_loop(tab_sh, acc_sh, src_v, dst_v, rows_v, sems, extra):
    def _sl(pos):
        return pl.ds(pos * B, B)

    def _fire(sup, buf):
        handles = []
        for r in range(K):
            handles.append(pltpu.async_copy(
                tab_sh.at[src_v.at[sup * K + r]],
                rows_v.at[_sl(buf * K + r)], sems[buf]))
        return handles

    handles = _fire(0, 0)
    for sup in range(SUP):
        nxt = _fire(sup + 1, (sup + 1) % 2) if sup + 1 < SUP else None
        for r in range(K):
            handles[r].wait()
            pltpu.sync_copy(rows_v.at[_sl((sup % 2) * K + r)],
                            acc_sh.at[dst_v.at[sup * K + r]], add=True)
        handles = nxt

    @pl.when(extra)
    def _():
        pltpu.async_copy(tab_sh.at[src_v.at[CW]], rows_v.at[_sl(0)],
                         sems[0]).wait()
        pltpu.sync_copy(rows_v.at[_sl(0)], acc_sh.at[dst_v.at[CW]], add=True)


def _pass1_body(yx_hbm, ei_hbm, a0_hbm, a1_hbm, wb_hbm,
                out_hbm, dstm_hbm,
                src_v, dst_v, a0_v, a1_v, wb_v, rows_v, zb_v,
                acc_sh, tab_sh, sem_a, sem_b):
    c = lax.axis_index("c")
    s = lax.axis_index("s")
    wid = s * NC + c
    base, extra = _worker_rows(wid)

    pltpu.sync_copy(yx_hbm.at[pl.ds(s * RPT, RPT), pl.ds(0, H)],
                    tab_sh.at[pl.ds(s * RPT, RPT)])

    @pl.when(c == 0)
    def _():
        pltpu.sync_copy(yx_hbm.at[pl.ds(s * RPT, RPT), pl.ds(H, H)],
                        acc_sh.at[pl.ds(s * RPT, RPT)])

    @pl.when(c != 0)
    def _():
        def _zero_row(i, carry):
            zb_v[i, :] = jnp.zeros((H,), jnp.float32)
            return carry
        lax.fori_loop(0, B, _zero_row, 0)
        for k in range(RPT // B):
            pltpu.sync_copy(zb_v, acc_sh.at[pl.ds(s * RPT + k * B, B)])

    _stage_edges(ei_hbm, 0, base, extra, src_v)
    _stage_edges(ei_hbm, 1, base, extra, dst_v)
    _stage_rows(a0_hbm, base, extra, a0_v)
    _stage_rows(a1_hbm, base, extra, a1_v)
    pltpu.sync_copy(wb_hbm, wb_v)
    wv = wb_v[4, :]
    w0 = wv[0]
    w1 = wv[1]

    def _mask_row(j, carry):
        for k in range(B // H):
            sl = pl.ds(k * H, H)
            me = w0 * a0_v[j, sl] + w1 * a1_v[j, sl]
            dst_v[j, sl] = jnp.where(me != 0.0, dst_v[j, sl], jnp.int32(N))
        return carry
    lax.fori_loop(0, CH, _mask_row, 0)

    pltpu.sync_copy(dst_v.at[pl.ds(0, CW)], dstm_hbm.at[pl.ds(base, CW)])

    @pl.when(extra)
    def _():
        pltpu.sync_copy(dst_v.at[pl.ds(CW, 1)],
                        dstm_hbm.at[pl.ds(base + CW, 1)])

    plsc.subcore_barrier()
    _edge_loop(tab_sh, acc_sh, src_v, dst_v, rows_v, (sem_a, sem_b), extra)
    plsc.subcore_barrier()
    pltpu.sync_copy(acc_sh.at[pl.ds(s * RPT, RPT)],
                    out_hbm.at[c, pl.ds(s * RPT, RPT)])


_pass1 = pl.kernel(
    _pass1_body,
    out_type=[jax.ShapeDtypeStruct((NC, NP, H), jnp.float32),
              jax.ShapeDtypeStruct((NR, B), jnp.int32)],
    mesh=plsc.VectorSubcoreMesh(core_axis_name="c", subcore_axis_name="s"),
    scratch_types=[
        pltpu.VMEM((CH, B), jnp.int32),
        pltpu.VMEM((CH, B), jnp.int32),
        pltpu.VMEM((CH, B), jnp.float32),
        pltpu.VMEM((CH, B), jnp.float32),
        pltpu.VMEM((8, H), jnp.float32),
        pltpu.VMEM((2 * K * B, H), jnp.float32),
        pltpu.VMEM((B, H), jnp.float32),
        pltpu.VMEM_SHARED((NP, H), jnp.float32),
        pltpu.VMEM_SHARED((NP, H), jnp.float32),
        pltpu.SemaphoreType.DMA,
        pltpu.SemaphoreType.DMA,
    ],
    compiler_params=pltpu.CompilerParams(use_tc_tiling_on_sc=False,
                                         needs_layout_passes=False),
)


def _pass2_body(p_hbm, ei_hbm, dstm_hbm, wb_hbm, out_hbm,
                src_v, dst_v, rows_v, p0_v, p1_v, y2_v, hr_v, q_v, wb_v,
                acc_sh, tab_sh, sem_a, sem_b):
    c = lax.axis_index("c")
    s = lax.axis_index("s")
    wid = s * NC + c
    base, extra = _worker_rows(wid)

    pltpu.sync_copy(p_hbm.at[0, pl.ds(s * RPT, RPT)], p0_v)
    pltpu.sync_copy(p_hbm.at[1, pl.ds(s * RPT, RPT)], p1_v)
    pltpu.sync_copy(wb_hbm, wb_v)
    b1v = wb_v[0, :]
    w2l = wb_v[1, :]
    w2r = wb_v[2, :]
    b2v = wb_v[3, :]

    def _group(g, carry):
        row_idx = g * H + lax.iota(jnp.int32, H)
        y2acc = jnp.zeros((H,), jnp.float32)
        hracc = jnp.zeros((H,), jnp.float32)
        for f in range(H):
            col_idx = jnp.full((H,), f, jnp.int32)
            c0 = plsc.load_gather(p0_v, [row_idx, col_idx])
            c1 = plsc.load_gather(p1_v, [row_idx, col_idx])
            hcol = jnp.maximum(c0 + c1 + b1v[f], 0.0)
            y2acc = y2acc + hcol * w2l[f]
            hracc = hracc + hcol * w2r[f]
        y2_v[pl.ds(g * H, H)] = y2acc
        hr_v[pl.ds(g * H, H)] = hracc + b2v
        return carry
    lax.fori_loop(0, NG, _group, 0)
    pltpu.sync_copy(y2_v, tab_sh.at[pl.ds(s * RPT, RPT)])

    def _zero_g(g, carry):
        y2_v[pl.ds(g * H, H)] = jnp.zeros((H,), jnp.float32)
        return carry
    lax.fori_loop(0, NG, _zero_g, 0)
    pltpu.sync_copy(y2_v, acc_sh.at[pl.ds(s * RPT, RPT)])

    _stage_edges(ei_hbm, 0, base, extra, src_v)
    _stage_rows(dstm_hbm, base, extra, dst_v)
    plsc.subcore_barrier()

    _edge_loop(tab_sh, acc_sh, src_v, dst_v, rows_v, (sem_a, sem_b), extra)
    plsc.subcore_barrier()

    pltpu.sync_copy(acc_sh.at[pl.ds(s * RPT, RPT)], q_v)

    def _combine(g, carry):
        sl = pl.ds(g * H, H)
        contrib = jnp.where(c == 0, hr_v[sl], jnp.zeros((H,), jnp.float32))
        q_v[sl] = q_v[sl] + contrib
        return carry
    lax.fori_loop(0, NG, _combine, 0)
    pltpu.sync_copy(q_v, out_hbm.at[c, pl.ds(s * RPT, RPT)])


_pass2 = pl.kernel(
    _pass2_body,
    out_type=jax.ShapeDtypeStruct((NC, NP), jnp.float32),
    mesh=plsc.VectorSubcoreMesh(core_axis_name="c", subcore_axis_name="s"),
    scratch_types=[
        pltpu.VMEM((CH, B), jnp.int32),
        pltpu.VMEM((CH, B), jnp.int32),
        pltpu.VMEM((2 * K * B,), jnp.float32),
        pltpu.VMEM((RPT, H), jnp.float32),
        pltpu.VMEM((RPT, H), jnp.float32),
        pltpu.VMEM((RPT,), jnp.float32),
        pltpu.VMEM((RPT,), jnp.float32),
        pltpu.VMEM((RPT,), jnp.float32),
        pltpu.VMEM((8, H), jnp.float32),
        pltpu.VMEM_SHARED((NP,), jnp.float32),
        pltpu.VMEM_SHARED((NP,), jnp.float32),
        pltpu.SemaphoreType.DMA,
        pltpu.SemaphoreType.DMA,
    ],
    compiler_params=pltpu.CompilerParams(use_tc_tiling_on_sc=False,
                                         needs_layout_passes=False),
)


def _final_body(o_ref, out_ref):
    out_ref[...] = o_ref[0] + o_ref[1]


_final = pl.pallas_call(
    _final_body,
    grid=(1,),
    in_specs=[pl.BlockSpec((NC, NP // 128, 128), lambda i: (0, 0, 0))],
    out_specs=pl.BlockSpec((NP // 128, 128), lambda i: (0, 0)),
    out_shape=jax.ShapeDtypeStruct((NP // 128, 128), jnp.float32),
)


def kernel(x, edge_index, A0, A1, w, W1l, W1r, b1, W2l, W2r, b2):
    ei3 = jnp.transpose(edge_index.reshape(2, NR, B), (1, 0, 2))
    a02 = A0.reshape(NR, B)
    a12 = A1.reshape(NR, B)
    xp = jnp.pad(x, ((0, NP - N), (0, 0)))
    w1c = jnp.concatenate(
        [W1l, W1r, jnp.zeros((D, D - 2 * H), jnp.float32)], axis=1)
    wb = jnp.stack([b1, W2l.reshape(H), W2r.reshape(H),
                    jnp.broadcast_to(b2, (H,)), jnp.pad(w, (0, H - 2)),
                    jnp.zeros((H,), jnp.float32), jnp.zeros((H,), jnp.float32),
                    jnp.zeros((H,), jnp.float32)])

    yx = _prep(xp, w1c)
    p, dstm = _pass1(yx, ei3, a02, a12, wb)
    o = _pass2(p, ei3, dstm, wb)
    out = _final(o.reshape(NC, NP // 128, 128))
    return out.reshape(NP, 1)[:N]


# speedup vs baseline: 1.2472x; 1.0164x over previous
import jax
import jax.numpy as jnp
from jax import lax
from jax.experimental import pallas as pl
from jax.experimental.pallas import tpu as pltpu
from jax.experimental.pallas import tpu_sc as plsc

N = 10000
D = 128
H = 16
E = 320000
NP = 10240
NC = 2
NS = 16
NW = NC * NS
B = 128
NR = E // B
K = 6
SUP = 13
CW = K * SUP
XW = NR - NW * CW
CH = CW + 1
RPT = NP // NS
NG = RPT // H


def _prep_body(x_ref, w_ref, yx_ref):
    x = x_ref[...]
    yx_ref[...] = jnp.dot(x, w_ref[...], preferred_element_type=jnp.float32)


_prep = pl.pallas_call(
    _prep_body,
    grid=(10,),
    in_specs=[
        pl.BlockSpec((NP // 10, D), lambda i: (i, 0)),
        pl.BlockSpec((D, D), lambda i: (0, 0)),
    ],
    out_specs=pl.BlockSpec((NP // 10, D), lambda i: (i, 0)),
    out_shape=jax.ShapeDtypeStruct((NP, D), jnp.float32),
)


def _worker_rows(wid):
    base = wid * CW + jnp.minimum(wid, XW)
    return base, wid < XW


def _stage_edges(ei_hbm, half, base, extra, v):
    pltpu.sync_copy(ei_hbm.at[pl.ds(base, CW), half], v.at[pl.ds(0, CW)])

    @pl.when(extra)
    def _():
        pltpu.sync_copy(ei_hbm.at[pl.ds(base + CW, 1), half],
                        v.at[pl.ds(CW, 1)])


def _stage_rows(src_hbm, base, extra, v):
    pltpu.sync_copy(src_hbm.at[pl.ds(base, CW)], v.at[pl.ds(0, CW)])

    @pl.when(extra)
    def _():
        pltpu.sync_copy(src_hbm.at[pl.ds(base + CW, 1)], v.at[pl.ds(CW, 1)])


def _edge_loop(tab_sh, acc_sh, src_v, dst_v, rows_v, sems, extra,
               mask_sup=None):
    def _sl(pos):
        return pl.ds(pos * B, B)

    def _fire(sup, buf):
        handles = []
        for r in range(K):
            handles.append(pltpu.async_copy(
                tab_sh.at[src_v.at[sup * K + r]],
                rows_v.at[_sl(buf * K + r)], sems[buf]))
        return handles

    handles = _fire(0, 0)
    if mask_sup is not None:
        mask_sup(0)
    for sup in range(SUP):
        nxt = _fire(sup + 1, (sup + 1) % 2) if sup + 1 < SUP else None
        if nxt is not None and mask_sup is not None:
            mask_sup(sup + 1)
        for r in range(K):
            handles[r].wait()
            pltpu.sync_copy(rows_v.at[_sl((sup % 2) * K + r)],
                            acc_sh.at[dst_v.at[sup * K + r]], add=True)
        handles = nxt

    @pl.when(extra)
    def _():
        pltpu.async_copy(tab_sh.at[src_v.at[CW]], rows_v.at[_sl(0)],
                         sems[0]).wait()
        pltpu.sync_copy(rows_v.at[_sl(0)], acc_sh.at[dst_v.at[CW]], add=True)


def _pass1_body(yx_hbm, ei_hbm, a0_hbm, a1_hbm, wb_hbm,
                out_hbm, dstm_hbm,
                src_v, dst_v, a0_v, a1_v, wb_v, rows_v, zb_v,
                acc_sh, tab_sh, sem_a, sem_b):
    c = lax.axis_index("c")
    s = lax.axis_index("s")
    wid = s * NC + c
    base, extra = _worker_rows(wid)

    pltpu.sync_copy(yx_hbm.at[pl.ds(s * RPT, RPT), pl.ds(0, H)],
                    tab_sh.at[pl.ds(s * RPT, RPT)])

    @pl.when(c == 0)
    def _():
        pltpu.sync_copy(yx_hbm.at[pl.ds(s * RPT, RPT), pl.ds(H, H)],
                        acc_sh.at[pl.ds(s * RPT, RPT)])

    @pl.when(c != 0)
    def _():
        def _zero_row(i, carry):
            zb_v[i, :] = jnp.zeros((H,), jnp.float32)
            return carry
        lax.fori_loop(0, B, _zero_row, 0)
        for k in range(RPT // B):
            pltpu.sync_copy(zb_v, acc_sh.at[pl.ds(s * RPT + k * B, B)])

    _stage_edges(ei_hbm, 0, base, extra, src_v)
    _stage_edges(ei_hbm, 1, base, extra, dst_v)
    _stage_rows(a0_hbm, base, extra, a0_v)
    _stage_rows(a1_hbm, base, extra, a1_v)
    pltpu.sync_copy(wb_hbm, wb_v)
    wv = wb_v[4, :]
    w0 = wv[0]
    w1 = wv[1]

    def _mask_row(j, carry):
        for k in range(B // H):
            sl = pl.ds(k * H, H)
            me = w0 * a0_v[j, sl] + w1 * a1_v[j, sl]
            dst_v[j, sl] = jnp.where(me != 0.0, dst_v[j, sl], jnp.int32(N))
        return carry

    def _mask_sup(sup):
        lax.fori_loop(sup * K, sup * K + K, _mask_row, 0)

    _mask_row(CW, 0)

    plsc.subcore_barrier()
    _edge_loop(tab_sh, acc_sh, src_v, dst_v, rows_v, (sem_a, sem_b), extra,
               mask_sup=_mask_sup)

    pltpu.sync_copy(dst_v.at[pl.ds(0, CW)], dstm_hbm.at[pl.ds(base, CW)])

    @pl.when(extra)
    def _():
        pltpu.sync_copy(dst_v.at[pl.ds(CW, 1)],
                        dstm_hbm.at[pl.ds(base + CW, 1)])

    plsc.subcore_barrier()
    pltpu.sync_copy(acc_sh.at[pl.ds(s * RPT, RPT)],
                    out_hbm.at[c, pl.ds(s * RPT, RPT)])


_pass1 = pl.kernel(
    _pass1_body,
    out_type=[jax.ShapeDtypeStruct((NC, NP, H), jnp.float32),
              jax.ShapeDtypeStruct((NR, B), jnp.int32)],
    mesh=plsc.VectorSubcoreMesh(core_axis_name="c", subcore_axis_name="s"),
    scratch_types=[
        pltpu.VMEM((CH, B), jnp.int32),
        pltpu.VMEM((CH, B), jnp.int32),
        pltpu.VMEM((CH, B), jnp.float32),
        pltpu.VMEM((CH, B), jnp.float32),
        pltpu.VMEM((8, H), jnp.float32),
        pltpu.VMEM((2 * K * B, H), jnp.float32),
        pltpu.VMEM((B, H), jnp.float32),
        pltpu.VMEM_SHARED((NP, H), jnp.float32),
        pltpu.VMEM_SHARED((NP, H), jnp.float32),
        pltpu.SemaphoreType.DMA,
        pltpu.SemaphoreType.DMA,
    ],
    compiler_params=pltpu.CompilerParams(use_tc_tiling_on_sc=False,
                                         needs_layout_passes=False),
)


def _pass2_body(p_hbm, ei_hbm, dstm_hbm, wb_hbm, out_hbm,
                src_v, dst_v, rows_v, p0_v, p1_v, y2_v, hr_v, q_v, wb_v,
                acc_sh, tab_sh, sem_a, sem_b):
    c = lax.axis_index("c")
    s = lax.axis_index("s")
    wid = s * NC + c
    base, extra = _worker_rows(wid)

    pltpu.sync_copy(p_hbm.at[0, pl.ds(s * RPT, RPT)], p0_v)
    pltpu.sync_copy(p_hbm.at[1, pl.ds(s * RPT, RPT)], p1_v)
    pltpu.sync_copy(wb_hbm, wb_v)
    b1v = wb_v[0, :]
    w2l = wb_v[1, :]
    w2r = wb_v[2, :]
    b2v = wb_v[3, :]

    def _group(g, carry):
        row_idx = g * H + lax.iota(jnp.int32, H)
        y2acc = jnp.zeros((H,), jnp.float32)
        hracc = jnp.zeros((H,), jnp.float32)
        for f in range(H):
            col_idx = jnp.full((H,), f, jnp.int32)
            c0 = plsc.load_gather(p0_v, [row_idx, col_idx])
            c1 = plsc.load_gather(p1_v, [row_idx, col_idx])
            hcol = jnp.maximum(c0 + c1 + b1v[f], 0.0)
            y2acc = y2acc + hcol * w2l[f]
            hracc = hracc + hcol * w2r[f]
        y2_v[pl.ds(g * H, H)] = y2acc
        hr_v[pl.ds(g * H, H)] = hracc + b2v
        return carry
    lax.fori_loop(0, NG, _group, 0)
    pltpu.sync_copy(y2_v, tab_sh.at[pl.ds(s * RPT, RPT)])

    def _zero_g(g, carry):
        y2_v[pl.ds(g * H, H)] = jnp.zeros((H,), jnp.float32)
        return carry
    lax.fori_loop(0, NG, _zero_g, 0)
    pltpu.sync_copy(y2_v, acc_sh.at[pl.ds(s * RPT, RPT)])

    _stage_edges(ei_hbm, 0, base, extra, src_v)
    _stage_rows(dstm_hbm, base, extra, dst_v)
    plsc.subcore_barrier()

    _edge_loop(tab_sh, acc_sh, src_v, dst_v, rows_v, (sem_a, sem_b), extra)
    plsc.subcore_barrier()

    pltpu.sync_copy(acc_sh.at[pl.ds(s * RPT, RPT)], q_v)

    def _combine(g, carry):
        sl = pl.ds(g * H, H)
        contrib = jnp.where(c == 0, hr_v[sl], jnp.zeros((H,), jnp.float32))
        q_v[sl] = q_v[sl] + contrib
        return carry
    lax.fori_loop(0, NG, _combine, 0)
    pltpu.sync_copy(q_v, out_hbm.at[c, pl.ds(s * RPT, RPT)])


_pass2 = pl.kernel(
    _pass2_body,
    out_type=jax.ShapeDtypeStruct((NC, NP), jnp.float32),
    mesh=plsc.VectorSubcoreMesh(core_axis_name="c", subcore_axis_name="s"),
    scratch_types=[
        pltpu.VMEM((CH, B), jnp.int32),
        pltpu.VMEM((CH, B), jnp.int32),
        pltpu.VMEM((2 * K * B,), jnp.float32),
        pltpu.VMEM((RPT, H), jnp.float32),
        pltpu.VMEM((RPT, H), jnp.float32),
        pltpu.VMEM((RPT,), jnp.float32),
        pltpu.VMEM((RPT,), jnp.float32),
        pltpu.VMEM((RPT,), jnp.float32),
        pltpu.VMEM((8, H), jnp.float32),
        pltpu.VMEM_SHARED((NP,), jnp.float32),
        pltpu.VMEM_SHARED((NP,), jnp.float32),
        pltpu.SemaphoreType.DMA,
        pltpu.SemaphoreType.DMA,
    ],
    compiler_params=pltpu.CompilerParams(use_tc_tiling_on_sc=False,
                                         needs_layout_passes=False),
)


def _final_body(o_ref, out_ref):
    out_ref[...] = o_ref[0] + o_ref[1]


_final = pl.pallas_call(
    _final_body,
    grid=(1,),
    in_specs=[pl.BlockSpec((NC, NP // 128, 128), lambda i: (0, 0, 0))],
    out_specs=pl.BlockSpec((NP // 128, 128), lambda i: (0, 0)),
    out_shape=jax.ShapeDtypeStruct((NP // 128, 128), jnp.float32),
)


def kernel(x, edge_index, A0, A1, w, W1l, W1r, b1, W2l, W2r, b2):
    ei3 = jnp.transpose(edge_index.reshape(2, NR, B), (1, 0, 2))
    a02 = A0.reshape(NR, B)
    a12 = A1.reshape(NR, B)
    xp = jnp.pad(x, ((0, NP - N), (0, 0)))
    w1c = jnp.concatenate(
        [W1l, W1r, jnp.zeros((D, D - 2 * H), jnp.float32)], axis=1)
    wb = jnp.stack([b1, W2l.reshape(H), W2r.reshape(H),
                    jnp.broadcast_to(b2, (H,)), jnp.pad(w, (0, H - 2)),
                    jnp.zeros((H,), jnp.float32), jnp.zeros((H,), jnp.float32),
                    jnp.zeros((H,), jnp.float32)])

    yx = _prep(xp, w1c)
    p, dstm = _pass1(yx, ei3, a02, a12, wb)
    o = _pass2(p, ei3, dstm, wb)
    out = _final(o.reshape(NC, NP // 128, 128))
    return out.reshape(NP, 1)[:N]
